# baseline (speedup 1.0000x reference)
import functools

import jax
import jax.numpy as jnp
from jax import lax
from jax.experimental import pallas as pl
from jax.experimental.pallas import tpu as pltpu
from jax.experimental.pallas import tpu_sc as plsc

BATCH = 16384
EMBED_K = 32
NUM_ROWS = 1000000
NUM_WORKERS = 32
ROWS_PER_WORKER = BATCH // NUM_WORKERS
GROUPS = ROWS_PER_WORKER // 16
LANE = 128
IDX_PAD = ROWS_PER_WORKER + 16


def _fire_half(table_hbm, idxv, half_ref, sem):
    copies = []
    for i in range(8):
        tcol = pl.multiple_of((idxv[i] // LANE) * LANE, LANE)
        copies.append(pltpu.async_copy(
            table_hbm.at[:, pl.ds(tcol, LANE)], half_ref.at[i], sem))
    return copies


def _sc_mf_body(uidx_hbm, vidx_hbm, wt_hbm, ht_hbm, out_hbm,
                uidx_v, vidx_v, blocks, u_slab0, u_slab1, out_v,
                sem0, sem1):
    cid = lax.axis_index("c")
    sid = lax.axis_index("s")
    wid = sid * 2 + cid
    base = wid * ROWS_PER_WORKER

    pltpu.sync_copy(
        uidx_hbm.at[pl.ds(base, ROWS_PER_WORKER)],
        uidx_v.at[pl.ds(0, ROWS_PER_WORKER)])
    pltpu.sync_copy(
        vidx_hbm.at[pl.ds(base, ROWS_PER_WORKER)],
        vidx_v.at[pl.ds(0, ROWS_PER_WORKER)])

    iota = lax.iota(jnp.int32, 16)
    slot8 = jnp.bitwise_and(iota, 7)
    low8 = iota < 8
    h0 = blocks.at[pl.ds(0, 8)]
    h1 = blocks.at[pl.ds(8, 8)]

    def extract(idxv, store_slab):
        cols = jnp.bitwise_and(idxv, LANE - 1)
        for k in range(EMBED_K):
            val = plsc.load_gather(
                blocks, [slot8 + store_slab[1] * 8,
                         jnp.full((16,), k, jnp.int32), cols])
            store_slab[0][k, :] = val

    def group_body(g, carry):
        u0 = uidx_v[pl.ds(g * 16, 16)]
        u1 = uidx_v[pl.ds(g * 16 + 8, 16)]
        v0 = vidx_v[pl.ds(g * 16, 16)]
        v1 = vidx_v[pl.ds(g * 16 + 8, 16)]

        cu0 = _fire_half(wt_hbm, u0, h0, sem0)
        cu1 = _fire_half(wt_hbm, u1, h1, sem1)
        for c in cu0:
            c.wait()
        extract(u0, (u_slab0, jnp.int32(0)))
        cv0 = _fire_half(ht_hbm, v0, h0, sem0)
        for c in cu1:
            c.wait()
        extract(u1, (u_slab1, jnp.int32(1)))
        cv1 = _fire_half(ht_hbm, v1, h1, sem1)

        for c in cv0:
            c.wait()
        vcols0 = jnp.bitwise_and(v0, LANE - 1)
        acc0 = jnp.zeros((16,), jnp.float32)
        for k in range(EMBED_K):
            v = plsc.load_gather(
                blocks, [slot8, jnp.full((16,), k, jnp.int32), vcols0])
            acc0 = acc0 + u_slab0[k, :] * v
        plsc.store_compressed(out_v.at[pl.ds(g * 16, 16)], acc0, mask=low8)

        for c in cv1:
            c.wait()
        vcols1 = jnp.bitwise_and(v1, LANE - 1)
        acc1 = jnp.zeros((16,), jnp.float32)
        for k in range(EMBED_K):
            v = plsc.load_gather(
                blocks, [slot8 + 8, jnp.full((16,), k, jnp.int32), vcols1])
            acc1 = acc1 + u_slab1[k, :] * v
        plsc.store_compressed(out_v.at[pl.ds(g * 16 + 8, 16)], acc1, mask=low8)
        return carry

    lax.fori_loop(0, GROUPS, group_body, 0)

    pltpu.sync_copy(
        out_v.at[pl.ds(0, ROWS_PER_WORKER)],
        out_hbm.at[pl.ds(base, ROWS_PER_WORKER)])


@functools.partial(
    pl.kernel,
    out_type=jax.ShapeDtypeStruct((BATCH,), jnp.float32),
    mesh=plsc.VectorSubcoreMesh(core_axis_name="c", subcore_axis_name="s"),
    compiler_params=pltpu.CompilerParams(
        needs_layout_passes=False, use_tc_tiling_on_sc=True),
    scratch_types=[
        pltpu.VMEM((IDX_PAD,), jnp.int32),
        pltpu.VMEM((IDX_PAD,), jnp.int32),
        pltpu.VMEM((16, EMBED_K, LANE), jnp.float32),
        pltpu.VMEM((EMBED_K, 16), jnp.float32),
        pltpu.VMEM((EMBED_K, 16), jnp.float32),
        pltpu.VMEM((IDX_PAD,), jnp.float32),
        pltpu.SemaphoreType.DMA,
        pltpu.SemaphoreType.DMA,
    ],
)
def _mf_sc(uidx_hbm, vidx_hbm, wt_hbm, ht_hbm, out_hbm,
           uidx_v, vidx_v, blocks, u_slab0, u_slab1, out_v, sem0, sem1):
    _sc_mf_body(uidx_hbm, vidx_hbm, wt_hbm, ht_hbm, out_hbm,
                uidx_v, vidx_v, blocks, u_slab0, u_slab1, out_v, sem0, sem1)


def kernel(x, W, H):
    uidx = x[:, 0].astype(jnp.int32)
    vidx = x[:, 1].astype(jnp.int32)
    return _mf_sc(uidx, vidx, W.T, H.T)


# speedup vs baseline: 1.0516x; 1.0516x over previous
import functools

import jax
import jax.numpy as jnp
from jax import lax
from jax.experimental import pallas as pl
from jax.experimental.pallas import tpu as pltpu
from jax.experimental.pallas import tpu_sc as plsc

BATCH = 16384
EMBED_K = 32
NUM_ROWS = 1000000
NUM_WORKERS = 32
ROWS_PER_WORKER = BATCH // NUM_WORKERS
GROUPS = ROWS_PER_WORKER // 16
LANE = 128


def _fetch_blocks(table_hbm, idxv, blocks, sem):
    copies = []
    for i in range(16):
        tcol = pl.multiple_of((idxv[i] // LANE) * LANE, LANE)
        copies.append(pltpu.async_copy(
            table_hbm.at[:, pl.ds(tcol, LANE)], blocks.at[i], sem))
    return copies


def _sc_mf_body(uidx_hbm, vidx_hbm, wt_hbm, ht_hbm, out_hbm,
                uidx_v, vidx_v, blocks, u_slab, out_v, sem):
    cid = lax.axis_index("c")
    sid = lax.axis_index("s")
    wid = sid * 2 + cid
    base = wid * ROWS_PER_WORKER

    pltpu.sync_copy(uidx_hbm.at[pl.ds(base, ROWS_PER_WORKER)], uidx_v)
    pltpu.sync_copy(vidx_hbm.at[pl.ds(base, ROWS_PER_WORKER)], vidx_v)

    iota = lax.iota(jnp.int32, 16)

    def group_body(g, carry):
        uvec = uidx_v[pl.ds(g * 16, 16)]
        ucols = jnp.bitwise_and(uvec, LANE - 1)
        for c in _fetch_blocks(wt_hbm, uvec, blocks, sem):
            c.wait()
        for k in range(EMBED_K):
            u = plsc.load_gather(
                blocks, [iota, jnp.full((16,), k, jnp.int32), ucols])
            u_slab[k, :] = u
        vvec = vidx_v[pl.ds(g * 16, 16)]
        vcols = jnp.bitwise_and(vvec, LANE - 1)
        for c in _fetch_blocks(ht_hbm, vvec, blocks, sem):
            c.wait()
        acc = jnp.zeros((16,), jnp.float32)
        for k in range(EMBED_K):
            v = plsc.load_gather(
                blocks, [iota, jnp.full((16,), k, jnp.int32), vcols])
            acc = acc + u_slab[k, :] * v
        out_v[pl.ds(g * 16, 16)] = acc
        return carry

    lax.fori_loop(0, GROUPS, group_body, 0)

    pltpu.sync_copy(out_v, out_hbm.at[pl.ds(base, ROWS_PER_WORKER)])


@functools.partial(
    pl.kernel,
    out_type=jax.ShapeDtypeStruct((BATCH,), jnp.float32),
    mesh=plsc.VectorSubcoreMesh(core_axis_name="c", subcore_axis_name="s"),
    compiler_params=pltpu.CompilerParams(
        needs_layout_passes=False, use_tc_tiling_on_sc=True),
    scratch_types=[
        pltpu.VMEM((ROWS_PER_WORKER,), jnp.int32),
        pltpu.VMEM((ROWS_PER_WORKER,), jnp.int32),
        pltpu.VMEM((16, EMBED_K, LANE), jnp.float32),
        pltpu.VMEM((EMBED_K, 16), jnp.float32),
        pltpu.VMEM((ROWS_PER_WORKER,), jnp.float32),
        pltpu.SemaphoreType.DMA,
    ],
)
def _mf_sc(uidx_hbm, vidx_hbm, wt_hbm, ht_hbm, out_hbm,
           uidx_v, vidx_v, blocks, u_slab, out_v, sem):
    _sc_mf_body(uidx_hbm, vidx_hbm, wt_hbm, ht_hbm, out_hbm,
                uidx_v, vidx_v, blocks, u_slab, out_v, sem)


def kernel(x, W, H):
    uidx = x[:, 0].astype(jnp.int32)
    vidx = x[:, 1].astype(jnp.int32)
    return _mf_sc(uidx, vidx, W.T, H.T)
